# baseline (speedup 1.0000x reference)
import functools

import jax
import jax.numpy as jnp
from jax import lax
from jax.experimental import pallas as pl
from jax.experimental.pallas import tpu as pltpu
from jax.experimental.pallas import tpu_sc as plsc

N = 10000
E = 320000
G = 16
IN_DIM = 128
HID = 64
EMB = 32

NC, NS = 2, 16
NW = NC * NS
CH = 128
NCHUNK = 80
EPT_PAD = NCHUNK * CH
E_PAD = EPT_PAD * NW
N_PAD = 10112
RPT = N_PAD // NS


def _make_prop(D):
    mesh = plsc.VectorSubcoreMesh(core_axis_name="c", subcore_axis_name="s")

    @functools.partial(
        pl.kernel,
        out_type=jax.ShapeDtypeStruct((NC, N_PAD, D), jnp.float32),
        mesh=mesh,
        compiler_params=pltpu.CompilerParams(use_tc_tiling_on_sc=False),
        scratch_types=[
            pltpu.VMEM_SHARED((N_PAD, D), jnp.float32),
            pltpu.VMEM((NCHUNK, CH), jnp.int32),
            pltpu.VMEM((NCHUNK, CH), jnp.int32),
            pltpu.VMEM((CH, D), jnp.float32),
            pltpu.VMEM((CH, D), jnp.float32),
            pltpu.VMEM((CH, D), jnp.float32),
            pltpu.VMEM((CH, D), jnp.float32),
            [pltpu.SemaphoreType.DMA] * 8,
        ],
    )
    def prop(g_hbm, src_hbm, dst_hbm, out_hbm, acc, isrc, idst,
             rows0, rows1, rows2, rows3, sems):
        c = lax.axis_index("c")
        s = lax.axis_index("s")
        wid = c * NS + s
        base = s * RPT
        rows = [rows0, rows1, rows2, rows3]
        gs = [sems[i] for i in range(4)]
        ss = [sems[i + 4] for i in range(4)]
        pltpu.sync_copy(src_hbm.at[wid], isrc)
        pltpu.sync_copy(dst_hbm.at[wid], idst)
        pltpu.sync_copy(g_hbm.at[pl.ds(base, RPT)], acc.at[pl.ds(base, RPT)])
        plsc.subcore_barrier()

        for u in range(4):
            pltpu.async_copy(g_hbm.at[isrc.at[u]], rows[u], gs[u])

        def body(jj, carry):
            j0 = jj * 4
            for u in range(4):
                j = j0 + u
                pltpu.make_async_copy(
                    g_hbm.at[isrc.at[j]], rows[u], gs[u]).wait()
                pltpu.async_copy(rows[u], acc.at[idst.at[j]], ss[u],
                                 add=True)

            @pl.when(jj < NCHUNK // 4 - 1)
            def _():
                for u in range(4):
                    j = j0 + u
                    pltpu.make_async_copy(
                        rows[u], acc.at[idst.at[j]], ss[u]).wait()
                    pltpu.async_copy(g_hbm.at[isrc.at[j + 4]], rows[u],
                                     gs[u])

            @pl.when(jj == NCHUNK // 4 - 1)
            def _():
                for u in range(4):
                    j = j0 + u
                    pltpu.make_async_copy(
                        rows[u], acc.at[idst.at[j]], ss[u]).wait()
            return carry

        lax.fori_loop(0, NCHUNK // 4, body, 0)
        plsc.subcore_barrier()
        pltpu.sync_copy(acc.at[pl.ds(base, RPT)],
                        out_hbm.at[c].at[pl.ds(base, RPT)])

    return prop


_prop32 = _make_prop(EMB)

CW = 4
NG = HID // CW
NREP = NW // NG
N_STR = N_PAD + 1
SCH = 57
CCH = 380
NIT = 19
ES = NW * SCH * CH
EC = NREP * CCH * CH
E_PAD_H = ES + EC


def _splat(v):
    return jnp.full((16,), v, jnp.int32)


def _make_hprop():
    mesh = plsc.VectorSubcoreMesh(core_axis_name="c", subcore_axis_name="s")

    @functools.partial(
        pl.kernel,
        out_type=[jax.ShapeDtypeStruct((NC, N_PAD, HID), jnp.float32),
                  jax.ShapeDtypeStruct((NW, CW, N_STR), jnp.float32)],
        mesh=mesh,
        compiler_params=pltpu.CompilerParams(
            use_tc_tiling_on_sc=False, needs_layout_passes=False),
        scratch_types=[
            pltpu.VMEM_SHARED((N_PAD, HID), jnp.float32),
            pltpu.VMEM((CW, N_STR), jnp.float32),
            pltpu.VMEM((SCH, CH), jnp.int32),
            pltpu.VMEM((SCH, CH), jnp.int32),
            pltpu.VMEM((CH, HID), jnp.float32),
            pltpu.VMEM((CH, HID), jnp.float32),
            pltpu.VMEM((CH, HID), jnp.float32),
            pltpu.VMEM((20, CH), jnp.int32),
            pltpu.VMEM((20, CH), jnp.int32),
            pltpu.VMEM((CH, CW), jnp.float32),
            pltpu.VMEM((CH, CW), jnp.float32),
            [pltpu.SemaphoreType.DMA] * 8,
        ],
    )
    def hprop(g_hbm, g8_hbm, ssrc, sdst, csrc, cdst, zt_hbm,
              outs_hbm, outt_hbm, accs, accc, isrc, idst,
              r0, r1, r2, cbs, cbd, cv0, cv1, sems):
        c = lax.axis_index("c")
        s = lax.axis_index("s")
        wid = c * NS + s
        r = wid // NG
        k = wid % NG
        base = s * RPT
        rows = [r0, r1, r2]
        gs = [sems[i] for i in range(3)]
        ss = [sems[i + 3] for i in range(3)]
        cg = [sems[6], sems[7]]
        cvs = [cv0, cv1]

        pltpu.sync_copy(ssrc.at[wid], isrc)
        pltpu.sync_copy(sdst.at[wid], idst)
        pltpu.sync_copy(zt_hbm, accc)
        pltpu.sync_copy(g_hbm.at[pl.ds(base, RPT)],
                        accs.at[pl.ds(base, RPT)])
        plsc.subcore_barrier()

        colv = lax.iota(jnp.int32, 16) & 3
        quarter = lax.shift_right_logical(lax.iota(jnp.int32, 16), 2)
        kv = jnp.full((16,), k, jnp.int32)
        pats = [_splat(4 * p) + quarter for p in range(4)]

        def compute_chunk(ci, vbuf):
            for gv in range(CH // 16):
                dstv = cbd[ci, pl.ds(gv * 16, 16)]
                gv16 = _splat(gv * 16)
                for p in range(4):
                    rowv = dstv.at[pats[p]].get(mode="promise_in_bounds")
                    vrow = pats[p] + gv16
                    valv = plsc.load_gather(vbuf, [vrow, colv])
                    plsc.addupdate_scatter(accc, [colv, rowv], valv)

        for q in range(3):
            pltpu.async_copy(g_hbm.at[isrc.at[q]], rows[q], gs[q])

        def body(jj, carry):
            j0 = jj * 3
            for q in range(3):
                j = j0 + q
                pltpu.make_async_copy(
                    g_hbm.at[isrc.at[j]], rows[q], gs[q]).wait()
                pltpu.async_copy(rows[q], accs.at[idst.at[j]], ss[q],
                                 add=True)

            cb = jj * 20
            pltpu.sync_copy(csrc.at[r].at[pl.ds(cb, 20)], cbs)
            pltpu.sync_copy(cdst.at[r].at[pl.ds(cb, 20)], cbd)

            def tx(t, carry3):
                ci = t // 8
                off = (t % 8) * 16
                v = cbs[ci, pl.ds(off, 16)]
                cbs[ci, pl.ds(off, 16)] = v * NG + kv
                return carry3

            lax.fori_loop(0, 160, tx, 0)

            pltpu.async_copy(g8_hbm.at[cbs.at[0]], cv0, cg[0])
            for t in range(10):
                c0 = 2 * t
                c1 = c0 + 1
                pltpu.async_copy(g8_hbm.at[cbs.at[c1]], cv1, cg[1])
                pltpu.make_async_copy(
                    g8_hbm.at[cbs.at[c0]], cv0, cg[0]).wait()
                compute_chunk(c0, cv0)
                if t < 9:
                    pltpu.async_copy(g8_hbm.at[cbs.at[c0 + 2]], cv0, cg[0])
                pltpu.make_async_copy(
                    g8_hbm.at[cbs.at[c1]], cv1, cg[1]).wait()
                compute_chunk(c1, cv1)

            @pl.when(jj < NIT - 1)
            def _():
                for q in range(3):
                    j = j0 + q
                    pltpu.make_async_copy(
                        rows[q], accs.at[idst.at[j]], ss[q]).wait()
                    pltpu.async_copy(g_hbm.at[isrc.at[j + 3]], rows[q],
                                     gs[q])

            @pl.when(jj == NIT - 1)
            def _():
                for q in range(3):
                    j = j0 + q
                    pltpu.make_async_copy(
                        rows[q], accs.at[idst.at[j]], ss[q]).wait()
            return carry

        lax.fori_loop(0, NIT, body, 0)
        plsc.subcore_barrier()
        pltpu.sync_copy(accs.at[pl.ds(base, RPT)],
                        outs_hbm.at[c].at[pl.ds(base, RPT)])
        pltpu.sync_copy(accc, outt_hbm.at[wid])

    return hprop


_hprop = _make_hprop()

def _make_deg():
    mesh = plsc.VectorSubcoreMesh(core_axis_name="c", subcore_axis_name="s")

    @functools.partial(
        pl.kernel,
        out_type=jax.ShapeDtypeStruct((NW, N_PAD), jnp.float32),
        mesh=mesh,
        compiler_params=pltpu.CompilerParams(
            use_tc_tiling_on_sc=False, needs_layout_passes=False),
        scratch_types=[
            pltpu.VMEM((N_PAD,), jnp.float32),
            pltpu.VMEM((NCHUNK, CH), jnp.int32),
        ],
    )
    def deg(dst_hbm, zeros_hbm, out_hbm, acc, idst):
        c = lax.axis_index("c")
        s = lax.axis_index("s")
        wid = c * NS + s
        pltpu.sync_copy(dst_hbm.at[wid], idst)
        pltpu.sync_copy(zeros_hbm, acc)
        onev = jnp.full((16,), 1.0, jnp.float32)

        def body(ci, carry):
            for u in range(CH // 16):
                dstv = idst[ci, pl.ds(u * 16, 16)]
                plsc.addupdate_scatter(acc, [dstv], onev)
            return carry

        lax.fori_loop(0, NCHUNK, body, 0)
        pltpu.sync_copy(acc, out_hbm.at[wid])

    return deg


_deg = _make_deg()


def _tc_first(x_ref, w_ref, dt_ref, g_ref, dinv_ref):
    deg = jnp.sum(dt_ref[...], axis=1, keepdims=True) + 1.0
    dinv = lax.rsqrt(deg)
    dinv_ref[...] = dinv
    g_ref[...] = dinv * jnp.dot(x_ref[...], w_ref[...],
                                preferred_element_type=jnp.float32)


def _tc_mid(a_ref, t_ref, g_ref, dinv_ref, b_ref, w_ref, gn_ref):
    p = a_ref[0] + a_ref[1] - g_ref[...]
    cols = []
    for k in range(NG):
        tk = t_ref[k]
        for rr in range(1, NREP):
            tk = tk + t_ref[rr * NG + k]
        cols.append(tk[:, :N_PAD])
    p = p + jnp.concatenate(cols, axis=0).T
    h = jnp.maximum(dinv_ref[...] * p + b_ref[...], 0.0)
    gn_ref[...] = dinv_ref[...] * jnp.dot(h, w_ref[...],
                                          preferred_element_type=jnp.float32)


def _tc_final(a0_ref, a1_ref, g_ref, dinv_ref, b_ref, batch_ref, out_ref):
    p = a0_ref[...] + a1_ref[...] - g_ref[...]
    h = dinv_ref[...] * p + b_ref[...]
    gid = lax.broadcasted_iota(jnp.int32, (G, N_PAD), 0)
    m = (batch_ref[...] == gid).astype(jnp.float32)
    sums = jnp.dot(m, h, preferred_element_type=jnp.float32)
    cnt = jnp.sum(m, axis=1, keepdims=True)
    out_ref[...] = sums / jnp.maximum(cnt, 1.0)


def kernel(x, edge_index, batch, W1, b1, W2, b2, W3, b3):
    f32 = jnp.float32
    src = edge_index[0].astype(jnp.int32)
    dst = edge_index[1].astype(jnp.int32)
    src3 = jnp.concatenate(
        [src, jnp.zeros((E_PAD - E,), jnp.int32)]).reshape(NW, NCHUNK, CH)
    dst3 = jnp.concatenate(
        [dst, jnp.full((E_PAD - E,), N, jnp.int32)]).reshape(NW, NCHUNK, CH)
    src_h = jnp.concatenate([src, jnp.zeros((E_PAD_H - E,), jnp.int32)])
    dst_h = jnp.concatenate([dst, jnp.full((E_PAD_H - E,), N, jnp.int32)])
    s_src = src_h[:ES].reshape(NW, SCH, CH)
    s_dst = dst_h[:ES].reshape(NW, SCH, CH)
    c_src = src_h[ES:].reshape(NREP, CCH, CH)
    c_dst = dst_h[ES:].reshape(NREP, CCH, CH)

    x_p = jnp.zeros((N_PAD, IN_DIM), f32).at[:N].set(x)
    batch_p = jnp.full((1, N_PAD), G, jnp.int32).at[0, :N].set(
        batch.astype(jnp.int32))
    z_n = jnp.zeros((N_PAD,), f32)
    z_t = jnp.zeros((CW, N_STR), f32)

    dparts = _deg(dst3, z_n)
    dT = dparts.T

    g1, dinv = pl.pallas_call(
        _tc_first,
        out_shape=[jax.ShapeDtypeStruct((N_PAD, HID), f32),
                   jax.ShapeDtypeStruct((N_PAD, 1), f32)],
    )(x_p, W1, dT)

    g1_8 = lax.optimization_barrier(g1.reshape(N_PAD * NG, CW))
    pa1, pt1 = _hprop(g1, g1_8, s_src, s_dst, c_src, c_dst, z_t)
    g2 = pl.pallas_call(
        _tc_mid,
        out_shape=jax.ShapeDtypeStruct((N_PAD, HID), f32),
    )(pa1, pt1, g1, dinv, b1.reshape(1, HID), W2)

    g2_8 = lax.optimization_barrier(g2.reshape(N_PAD * NG, CW))
    pa2, pt2 = _hprop(g2, g2_8, s_src, s_dst, c_src, c_dst, z_t)
    g3 = pl.pallas_call(
        _tc_mid,
        out_shape=jax.ShapeDtypeStruct((N_PAD, EMB), f32),
    )(pa2, pt2, g2, dinv, b2.reshape(1, HID), W3)

    p3 = _prop32(g3, src3, dst3)
    out = pl.pallas_call(
        _tc_final,
        out_shape=jax.ShapeDtypeStruct((G, EMB), f32),
    )(p3[0], p3[1], g3, dinv, b3.reshape(1, EMB), batch_p)
    return out


# speedup vs baseline: 3.6404x; 3.6404x over previous
import functools

import jax
import jax.numpy as jnp
from jax import lax
from jax.experimental import pallas as pl
from jax.experimental.pallas import tpu as pltpu
from jax.experimental.pallas import tpu_sc as plsc

N = 10000
E = 320000
G = 16
IN_DIM = 128
HID = 64
EMB = 32

NC, NS = 2, 16
NW = NC * NS
CH = 128
NCHUNK = 80
EPT_PAD = NCHUNK * CH
E_PAD = EPT_PAD * NW
N_PAD = 10112
RPT = N_PAD // NS


def _make_prop(D):
    mesh = plsc.VectorSubcoreMesh(core_axis_name="c", subcore_axis_name="s")

    @functools.partial(
        pl.kernel,
        out_type=jax.ShapeDtypeStruct((NC, N_PAD, D), jnp.float32),
        mesh=mesh,
        compiler_params=pltpu.CompilerParams(use_tc_tiling_on_sc=False),
        scratch_types=[
            pltpu.VMEM_SHARED((N_PAD, D), jnp.float32),
            pltpu.VMEM((NCHUNK, CH), jnp.int32),
            pltpu.VMEM((NCHUNK, CH), jnp.int32),
            pltpu.VMEM((CH, D), jnp.float32),
            pltpu.VMEM((CH, D), jnp.float32),
            pltpu.VMEM((CH, D), jnp.float32),
            pltpu.VMEM((CH, D), jnp.float32),
            [pltpu.SemaphoreType.DMA] * 8,
        ],
    )
    def prop(g_hbm, src_hbm, dst_hbm, out_hbm, acc, isrc, idst,
             rows0, rows1, rows2, rows3, sems):
        c = lax.axis_index("c")
        s = lax.axis_index("s")
        wid = c * NS + s
        base = s * RPT
        rows = [rows0, rows1, rows2, rows3]
        gs = [sems[i] for i in range(4)]
        ss = [sems[i + 4] for i in range(4)]
        pltpu.sync_copy(src_hbm.at[wid], isrc)
        pltpu.sync_copy(dst_hbm.at[wid], idst)
        pltpu.sync_copy(g_hbm.at[pl.ds(base, RPT)], acc.at[pl.ds(base, RPT)])
        plsc.subcore_barrier()

        for u in range(4):
            pltpu.async_copy(g_hbm.at[isrc.at[u]], rows[u], gs[u])

        def body(jj, carry):
            j0 = jj * 4
            for u in range(4):
                j = j0 + u
                pltpu.make_async_copy(
                    g_hbm.at[isrc.at[j]], rows[u], gs[u]).wait()
                pltpu.async_copy(rows[u], acc.at[idst.at[j]], ss[u],
                                 add=True)

            @pl.when(jj < NCHUNK // 4 - 1)
            def _():
                for u in range(4):
                    j = j0 + u
                    pltpu.make_async_copy(
                        rows[u], acc.at[idst.at[j]], ss[u]).wait()
                    pltpu.async_copy(g_hbm.at[isrc.at[j + 4]], rows[u],
                                     gs[u])

            @pl.when(jj == NCHUNK // 4 - 1)
            def _():
                for u in range(4):
                    j = j0 + u
                    pltpu.make_async_copy(
                        rows[u], acc.at[idst.at[j]], ss[u]).wait()
            return carry

        lax.fori_loop(0, NCHUNK // 4, body, 0)
        plsc.subcore_barrier()
        pltpu.sync_copy(acc.at[pl.ds(base, RPT)],
                        out_hbm.at[c].at[pl.ds(base, RPT)])

    return prop


_prop64 = _make_prop(HID)
_prop32 = _make_prop(EMB)

def _make_deg():
    mesh = plsc.VectorSubcoreMesh(core_axis_name="c", subcore_axis_name="s")

    @functools.partial(
        pl.kernel,
        out_type=jax.ShapeDtypeStruct((NW, N_PAD), jnp.float32),
        mesh=mesh,
        compiler_params=pltpu.CompilerParams(
            use_tc_tiling_on_sc=False, needs_layout_passes=False),
        scratch_types=[
            pltpu.VMEM((N_PAD,), jnp.float32),
            pltpu.VMEM((NCHUNK, CH), jnp.int32),
        ],
    )
    def deg(dst_hbm, zeros_hbm, out_hbm, acc, idst):
        c = lax.axis_index("c")
        s = lax.axis_index("s")
        wid = c * NS + s
        pltpu.sync_copy(dst_hbm.at[wid], idst)
        pltpu.sync_copy(zeros_hbm, acc)
        onev = jnp.full((16,), 1.0, jnp.float32)

        def body(ci, carry):
            for u in range(CH // 16):
                dstv = idst[ci, pl.ds(u * 16, 16)]
                plsc.addupdate_scatter(acc, [dstv], onev)
            return carry

        lax.fori_loop(0, NCHUNK, body, 0)
        pltpu.sync_copy(acc, out_hbm.at[wid])

    return deg


_deg = _make_deg()


def _tc_first(x_ref, w_ref, dt_ref, g_ref, dinv_ref):
    deg = jnp.sum(dt_ref[...], axis=1, keepdims=True) + 1.0
    dinv = lax.rsqrt(deg)
    dinv_ref[...] = dinv
    g_ref[...] = dinv * jnp.dot(x_ref[...], w_ref[...],
                                preferred_element_type=jnp.float32)


def _tc_mid(a0_ref, a1_ref, g_ref, dinv_ref, b_ref, w_ref, gn_ref):
    p = a0_ref[...] + a1_ref[...] - g_ref[...]
    h = jnp.maximum(dinv_ref[...] * p + b_ref[...], 0.0)
    gn_ref[...] = dinv_ref[...] * jnp.dot(h, w_ref[...],
                                          preferred_element_type=jnp.float32)


def _tc_final(a0_ref, a1_ref, g_ref, dinv_ref, b_ref, batch_ref, out_ref):
    p = a0_ref[...] + a1_ref[...] - g_ref[...]
    h = dinv_ref[...] * p + b_ref[...]
    gid = lax.broadcasted_iota(jnp.int32, (G, N_PAD), 0)
    m = (batch_ref[...] == gid).astype(jnp.float32)
    sums = jnp.dot(m, h, preferred_element_type=jnp.float32)
    cnt = jnp.sum(m, axis=1, keepdims=True)
    out_ref[...] = sums / jnp.maximum(cnt, 1.0)


def kernel(x, edge_index, batch, W1, b1, W2, b2, W3, b3):
    f32 = jnp.float32
    src = edge_index[0].astype(jnp.int32)
    dst = edge_index[1].astype(jnp.int32)
    src3 = jnp.concatenate(
        [src, jnp.zeros((E_PAD - E,), jnp.int32)]).reshape(NW, NCHUNK, CH)
    dst3 = jnp.concatenate(
        [dst, jnp.full((E_PAD - E,), N, jnp.int32)]).reshape(NW, NCHUNK, CH)

    x_p = jnp.zeros((N_PAD, IN_DIM), f32).at[:N].set(x)
    batch_p = jnp.full((1, N_PAD), G, jnp.int32).at[0, :N].set(
        batch.astype(jnp.int32))
    z_n = jnp.zeros((N_PAD,), f32)

    dparts = _deg(dst3, z_n)
    dT = dparts.T

    g1, dinv = pl.pallas_call(
        _tc_first,
        out_shape=[jax.ShapeDtypeStruct((N_PAD, HID), f32),
                   jax.ShapeDtypeStruct((N_PAD, 1), f32)],
    )(x_p, W1, dT)

    p1 = _prop64(g1, src3, dst3)
    g2 = pl.pallas_call(
        _tc_mid,
        out_shape=jax.ShapeDtypeStruct((N_PAD, HID), f32),
    )(p1[0], p1[1], g1, dinv, b1.reshape(1, HID), W2)

    p2 = _prop64(g2, src3, dst3)
    g3 = pl.pallas_call(
        _tc_mid,
        out_shape=jax.ShapeDtypeStruct((N_PAD, EMB), f32),
    )(p2[0], p2[1], g2, dinv, b2.reshape(1, HID), W3)

    p3 = _prop32(g3, src3, dst3)
    out = pl.pallas_call(
        _tc_final,
        out_shape=jax.ShapeDtypeStruct((G, EMB), f32),
    )(p3[0], p3[1], g3, dinv, b3.reshape(1, EMB), batch_p)
    return out
